# Initial kernel scaffold; baseline (speedup 1.0000x reference)
import jax
import jax.numpy as jnp
from jax.experimental import pallas as pl


def kernel(x, w_down, expert_pos_embed, global_w, global_b, in_proj_w, in_proj_b, out_proj_w, out_proj_b, norm_g, norm_b, q_w1, q_b1, q_ln_g, q_ln_b, q_w2, q_b2, k_w, k_b, w_up):
    raise NotImplementedError("write your pallas kernel here")


# speedup vs baseline: 2.9313x; 2.9313x over previous
import math

import jax
import jax.numpy as jnp
from jax.experimental import pallas as pl
from jax.experimental.pallas import tpu as pltpu

B, T, DM = 2, 2048, 1024
E, DL, TOPK, H = 8, 64, 3, 4
N = B * T
S = E + 1
DH = DL // H
BLK = 512
NBLK = N // BLK
TPB = T // BLK

F32 = jnp.float32
BF16 = jnp.bfloat16


def _gelu_exact(x):
    return 0.5 * x * (1.0 + jax.lax.erf(x * (1.0 / math.sqrt(2.0))))


def _b16(x):
    return x.astype(BF16)


def _dotb(a, b):
    return jnp.dot(a, b, preferred_element_type=F32)


def _dotx(a, b):
    return jnp.dot(a, b, preferred_element_type=F32,
                   precision=jax.lax.Precision.HIGHEST)


def _ctx_kernel(x_ref, gw_ref, gb_ref, qw1_ref, qb1_ref, qlg_ref, qlb_ref,
                qw2_ref, qb2_ref, gc_ref, rq_ref):
    xm = jnp.mean(x_ref[0], axis=0, keepdims=True)
    gc = _dotb(_b16(xm), gw_ref[...]) + gb_ref[...]
    gc_ref[0] = gc
    z = _dotb(_b16(gc), qw1_ref[...]) + qb1_ref[...]
    m = jnp.mean(z, axis=1, keepdims=True)
    v = jnp.mean((z - m) ** 2, axis=1, keepdims=True)
    z = (z - m) / jnp.sqrt(v + 1e-5) * qlg_ref[...] + qlb_ref[...]
    z = _gelu_exact(z)
    rq_ref[0] = _dotb(_b16(z), qw2_ref[...]) + qb2_ref[...]


def _main_kernel(x_ref, wdt_ref, pos_ref, gc_ref, rq_ref, ipw_ref, ipb_ref,
                 opw_ref, opb_ref, ng_ref, nb_ref, kwt_ref, kb_ref,
                 g2_ref, p36_ref, gexp_ref, wup_ref,
                 out_ref, aux_ref, sp_acc, sl_acc):
    step = pl.program_id(0)
    ef = _dotb(_b16(x_ref[...]), wdt_ref[...])
    gc = jnp.broadcast_to(gc_ref[0], (BLK, DL))
    seq = [gc] + [ef[:, e * DL:(e + 1) * DL] + pos_ref[:, e * DL:(e + 1) * DL]
                  for e in range(E)]
    ipw = ipw_ref[...]
    ipb = ipb_ref[...]
    qs, ks, vs = [], [], []
    for j in range(S):
        qkv = _dotb(_b16(seq[j]), ipw) + ipb
        qs.append(qkv[:, :DL])
        ks.append(qkv[:, DL:2 * DL])
        vs.append(_b16(qkv[:, 2 * DL:]).astype(F32))
    kcat = _b16(jnp.concatenate(ks, axis=1)).astype(F32)
    g2 = g2_ref[...]
    p36 = p36_ref[...]
    opw = opw_ref[...]
    opb = opb_ref[...]
    ng = ng_ref[...]
    nb = nb_ref[...]
    kwt = kwt_ref[...]
    kb = kb_ref[...]
    rq = jnp.broadcast_to(rq_ref[0], (BLK, DL))
    logits = []
    for i in range(1, S):
        qi = _b16(qs[i]).astype(F32)
        qt = jnp.concatenate([qi] * S, axis=1)
        sc = _dotx(kcat * qt, g2)
        sc = sc * (1.0 / math.sqrt(DH))
        a_parts = []
        for h in range(H):
            sch = sc[:, h * S:(h + 1) * S]
            mh = jnp.max(sch, axis=1, keepdims=True)
            exh = jnp.exp(sch - mh)
            a_parts.append(exh / jnp.sum(exh, axis=1, keepdims=True))
        a = jnp.concatenate(a_parts, axis=1)
        ab = _dotb(_b16(a), p36)
        ao = ab[:, :DL] * vs[0]
        for j in range(1, S):
            ao = ao + ab[:, j * DL:(j + 1) * DL] * vs[j]
        ao = _dotb(_b16(ao), opw) + opb
        hres = ao + seq[i]
        m = jnp.mean(hres, axis=1, keepdims=True)
        v = jnp.mean((hres - m) ** 2, axis=1, keepdims=True)
        inter = (hres - m) / jnp.sqrt(v + 1e-5) * ng + nb
        rk = _dotb(_b16(inter), kwt) + kb
        logits.append(jnp.sum(rk * rq, axis=1, keepdims=True) * (1.0 / math.sqrt(DL)))
    lg = jnp.concatenate(logits, axis=1)
    lg = lg - jnp.max(lg, axis=1, keepdims=True)
    pe = jnp.exp(lg)
    probs = pe / jnp.sum(pe, axis=1, keepdims=True)
    lane = jax.lax.broadcasted_iota(jnp.int32, (BLK, E), 1)
    cur = probs
    mask = jnp.zeros((BLK, E), F32)
    for _ in range(TOPK):
        mx = jnp.max(cur, axis=1, keepdims=True)
        eq = cur == mx
        cand = jnp.where(eq, lane, E)
        mi = jnp.min(cand, axis=1, keepdims=True)
        hit = lane == mi
        mask = mask + hit.astype(F32)
        cur = jnp.where(hit, -1.0, cur)
    wsel = probs * mask
    wt = wsel / jnp.sum(wsel, axis=1, keepdims=True)

    @pl.when(step == 0)
    def _():
        sp_acc[...] = jnp.zeros_like(sp_acc)
        sl_acc[...] = jnp.zeros_like(sl_acc)

    sp_acc[...] += jnp.sum(probs, axis=0, keepdims=True)
    sl_acc[...] += jnp.sum(mask, axis=0, keepdims=True)
    aux_ref[...] = (jnp.sum(sp_acc[...] * sl_acc[...]) * (E / (N * N))).reshape(1, 1)

    act = _b16(_gelu_exact(ef)).astype(F32)
    u = act * _dotx(wt, gexp_ref[...])
    out_ref[...] = _dotb(_b16(u), wup_ref[...])


def kernel(x, w_down, expert_pos_embed, global_w, global_b, in_proj_w,
           in_proj_b, out_proj_w, out_proj_b, norm_g, norm_b, q_w1, q_b1,
           q_ln_g, q_ln_b, q_w2, q_b2, k_w, k_b, w_up):
    xf = x.reshape(N, DM)
    wdt = w_down.T.astype(BF16)
    pos = expert_pos_embed.reshape(1, E * DL)
    ipw_t = in_proj_w.T.astype(BF16)
    opw_t = out_proj_w.T.astype(BF16)
    kwt = k_w.T.astype(BF16)
    gwt = global_w.T.astype(BF16)
    qw1t = q_w1.T.astype(BF16)
    qw2t = q_w2.T.astype(BF16)
    wup = w_up.reshape(E * DL, DM).astype(BF16)

    def row(vec):
        return vec.reshape(1, -1)

    r576 = jnp.arange(S * DL)
    jj = r576 // DL
    d = r576 % DL
    a36 = jnp.arange(H * S)
    cols36 = (d // DH) * S + jj
    G2 = (cols36[:, None] == a36[None, :]).astype(F32)
    P36 = (((a36 % S)[:, None] == jj[None, :])
           & ((a36 // S)[:, None] == (d[None, :] // DH))).astype(BF16)
    Gexp = (jnp.arange(E)[:, None] == (jnp.arange(E * DL)[None, :] // DL)).astype(F32)

    gc, rq = pl.pallas_call(
        _ctx_kernel,
        grid=(B,),
        in_specs=[
            pl.BlockSpec((1, T, DM), lambda i: (i, 0, 0)),
            pl.BlockSpec((DM, DL), lambda i: (0, 0)),
            pl.BlockSpec((1, DL), lambda i: (0, 0)),
            pl.BlockSpec((DL, DL), lambda i: (0, 0)),
            pl.BlockSpec((1, DL), lambda i: (0, 0)),
            pl.BlockSpec((1, DL), lambda i: (0, 0)),
            pl.BlockSpec((1, DL), lambda i: (0, 0)),
            pl.BlockSpec((DL, DL), lambda i: (0, 0)),
            pl.BlockSpec((1, DL), lambda i: (0, 0)),
        ],
        out_specs=[
            pl.BlockSpec((1, 1, DL), lambda i: (i, 0, 0)),
            pl.BlockSpec((1, 1, DL), lambda i: (i, 0, 0)),
        ],
        out_shape=[
            jax.ShapeDtypeStruct((B, 1, DL), F32),
            jax.ShapeDtypeStruct((B, 1, DL), F32),
        ],
        compiler_params=pltpu.CompilerParams(
            dimension_semantics=("arbitrary",)),
    )(x, gwt, row(global_b), qw1t, row(q_b1), row(q_ln_g), row(q_ln_b),
      qw2t, row(q_b2))

    out, aux = pl.pallas_call(
        _main_kernel,
        grid=(NBLK,),
        in_specs=[
            pl.BlockSpec((BLK, DM), lambda i: (i, 0)),
            pl.BlockSpec((DM, E * DL), lambda i: (0, 0)),
            pl.BlockSpec((1, E * DL), lambda i: (0, 0)),
            pl.BlockSpec((1, 1, DL), lambda i: (i // TPB, 0, 0)),
            pl.BlockSpec((1, 1, DL), lambda i: (i // TPB, 0, 0)),
            pl.BlockSpec((DL, 3 * DL), lambda i: (0, 0)),
            pl.BlockSpec((1, 3 * DL), lambda i: (0, 0)),
            pl.BlockSpec((DL, DL), lambda i: (0, 0)),
            pl.BlockSpec((1, DL), lambda i: (0, 0)),
            pl.BlockSpec((1, DL), lambda i: (0, 0)),
            pl.BlockSpec((1, DL), lambda i: (0, 0)),
            pl.BlockSpec((DL, DL), lambda i: (0, 0)),
            pl.BlockSpec((1, DL), lambda i: (0, 0)),
            pl.BlockSpec((S * DL, H * S), lambda i: (0, 0)),
            pl.BlockSpec((H * S, S * DL), lambda i: (0, 0)),
            pl.BlockSpec((E, E * DL), lambda i: (0, 0)),
            pl.BlockSpec((E * DL, DM), lambda i: (0, 0)),
        ],
        out_specs=[
            pl.BlockSpec((BLK, DM), lambda i: (i, 0)),
            pl.BlockSpec((1, 1), lambda i: (0, 0)),
        ],
        out_shape=[
            jax.ShapeDtypeStruct((N, DM), F32),
            jax.ShapeDtypeStruct((1, 1), F32),
        ],
        scratch_shapes=[
            pltpu.VMEM((1, E), F32),
            pltpu.VMEM((1, E), F32),
        ],
        compiler_params=pltpu.CompilerParams(
            dimension_semantics=("arbitrary",)),
    )(xf, wdt, pos, gc, rq, ipw_t, row(in_proj_b), opw_t, row(out_proj_b),
      row(norm_g), row(norm_b), kwt, row(k_b), G2, P36, Gexp, wup)

    return out.reshape(B, T, DM), aux[0, 0]


# speedup vs baseline: 6.3451x; 2.1646x over previous
import functools
import math

import jax
import jax.numpy as jnp
from jax import lax
from jax.experimental import pallas as pl
from jax.experimental.pallas import tpu as pltpu
from jax.experimental.pallas import tpu_sc as plsc

B, T, DM = 2, 2048, 1024
E, DL, TOPK, H = 8, 64, 3, 4
N = B * T
S = E + 1
DH = DL // H
BLK = 512
NBLK = N // BLK
TPB = T // BLK

F32 = jnp.float32
BF16 = jnp.bfloat16

SC_NC, SC_NS, SC_NL = 2, 16, 16
NW = SC_NC * SC_NS
TPW = N // NW
SC_CH = TPW // SC_NL


def _gelu_exact(x):
    return 0.5 * x * (1.0 + jax.lax.erf(x * (1.0 / math.sqrt(2.0))))


def _b16(x):
    return x.astype(BF16)


def _dotb(a, b):
    return jnp.dot(a, b, preferred_element_type=F32)


def _dotx(a, b):
    return jnp.dot(a, b, preferred_element_type=F32,
                   precision=jax.lax.Precision.HIGHEST)


def _doth(a, b16_mat):
    hi = _b16(a)
    lo = _b16(a - hi.astype(F32))
    return _dotb(hi, b16_mat) + _dotb(lo, b16_mat)


def _ctx_kernel(x_ref, gw_ref, gb_ref, qw1_ref, qb1_ref, qlg_ref, qlb_ref,
                qw2_ref, qb2_ref, gc_ref, rq_ref):
    xm = jnp.mean(x_ref[0], axis=0, keepdims=True)
    gc = _dotb(_b16(xm), gw_ref[...]) + gb_ref[...]
    gc_ref[0] = gc
    z = _dotb(_b16(gc), qw1_ref[...]) + qb1_ref[...]
    m = jnp.mean(z, axis=1, keepdims=True)
    v = jnp.mean((z - m) ** 2, axis=1, keepdims=True)
    z = (z - m) / jnp.sqrt(v + 1e-5) * qlg_ref[...] + qlb_ref[...]
    z = _gelu_exact(z)
    rq_ref[0] = _dotb(_b16(z), qw2_ref[...]) + qb2_ref[...]


def _main_kernel(x_ref, wdt_ref, pos_ref, gc_ref, rq_ref, ipw_ref, ipb_ref,
                 opw_ref, opb_ref, ng_ref, nb_ref, kwt_ref, kb_ref,
                 g2_ref, gss_ref, p36_ref,
                 ef_ref, lgt_ref):
    ef = _dotb(_b16(x_ref[...]), wdt_ref[...])
    gc = jnp.broadcast_to(gc_ref[0], (BLK, DL))
    seq = [gc] + [ef[:, e * DL:(e + 1) * DL] + pos_ref[:, e * DL:(e + 1) * DL]
                  for e in range(E)]
    seq_stack = jnp.concatenate(seq, axis=0)
    qkv = _dotb(_b16(seq_stack), ipw_ref[...]) + ipb_ref[...]
    qs = qkv[:, :DL]
    kv16 = _b16(qkv[:, DL:]).astype(F32)
    vs = [kv16[j * BLK:(j + 1) * BLK, DL:] for j in range(S)]
    kcat = jnp.concatenate(
        [kv16[j * BLK:(j + 1) * BLK, :DL] for j in range(S)], axis=1)
    g2 = g2_ref[...]
    p36 = p36_ref[...]
    rq = jnp.broadcast_to(rq_ref[0], ((S - 1) * BLK, DL))
    sc_parts = []
    for i in range(1, S):
        qi = _b16(qs[i * BLK:(i + 1) * BLK, :]).astype(F32)
        qt = jnp.concatenate([qi] * S, axis=1)
        sc_parts.append(_doth(kcat * qt, g2))
    sc = jnp.concatenate(sc_parts, axis=0) * (1.0 / math.sqrt(DH))
    mh = jnp.concatenate(
        [jnp.broadcast_to(
            jnp.max(sc[:, h * S:(h + 1) * S], axis=1, keepdims=True),
            ((S - 1) * BLK, S)) for h in range(H)], axis=1)
    ex = jnp.exp(sc - mh)
    a = ex / _dotx(ex, gss_ref[...])
    ab = _dotb(_b16(a), p36)
    ao_parts = []
    for i in range(1, S):
        abi = ab[(i - 1) * BLK:i * BLK, :]
        ao = abi[:, :DL] * vs[0]
        for j in range(1, S):
            ao = ao + abi[:, j * DL:(j + 1) * DL] * vs[j]
        ao_parts.append(ao)
    ao = jnp.concatenate(ao_parts, axis=0)
    ao = _dotb(_b16(ao), opw_ref[...]) + opb_ref[...]
    hres = ao + seq_stack[BLK:, :]
    m = jnp.mean(hres, axis=1, keepdims=True)
    v = jnp.mean((hres - m) ** 2, axis=1, keepdims=True)
    inter = (hres - m) / jnp.sqrt(v + 1e-5) * ng_ref[...] + nb_ref[...]
    rk = _dotb(_b16(inter), kwt_ref[...]) + kb_ref[...]
    lg_col = jnp.sum(rk * rq, axis=1, keepdims=True) * (1.0 / math.sqrt(DL))
    lg = jnp.concatenate(
        [lg_col[(i - 1) * BLK:i * BLK, :] for i in range(1, S)], axis=1)
    ef_ref[...] = ef
    lgt_ref[...] = lg.T


def _sc_route(lgt_hbm, wt_hbm, st_hbm, lg_v, wt_v, st_v):
    wid = lax.axis_index("s") * SC_NC + lax.axis_index("c")
    base = wid * TPW
    for e in range(E):
        pltpu.sync_copy(lgt_hbm.at[pl.ds(e * N + base, TPW)],
                        lg_v.at[pl.ds(e * TPW, TPW)])
    sp = [jnp.zeros((SC_NL,), F32) for _ in range(E)]
    sl = [jnp.zeros((SC_NL,), F32) for _ in range(E)]
    for c in range(SC_CH):
        lg = [lg_v[pl.ds(e * TPW + c * SC_NL, SC_NL)] for e in range(E)]
        m = lg[0]
        for e in range(1, E):
            m = jnp.maximum(m, lg[e])
        ex = [jnp.exp(lg[e] - m) for e in range(E)]
        s = ex[0]
        for e in range(1, E):
            s = s + ex[e]
        pr = [ex[e] / s for e in range(E)]
        cur = list(pr)
        msk = [jnp.zeros((SC_NL,), F32) for _ in range(E)]
        for _ in range(TOPK):
            mx = cur[0]
            for e in range(1, E):
                mx = jnp.maximum(mx, cur[e])
            found = jnp.zeros((SC_NL,), F32)
            for e in range(E):
                sel = jnp.where(cur[e] == mx, 1.0 - found, 0.0)
                msk[e] = msk[e] + sel
                cur[e] = cur[e] - sel * (cur[e] + 1.0)
                found = found + sel
        w = [pr[e] * msk[e] for e in range(E)]
        tws = w[0]
        for e in range(1, E):
            tws = tws + w[e]
        for e in range(E):
            wt_v[pl.ds(e * TPW + c * SC_NL, SC_NL)] = w[e] / tws
            sp[e] = sp[e] + pr[e]
            sl[e] = sl[e] + msk[e]
    for e in range(E):
        st_v[pl.ds(e * SC_NL, SC_NL)] = sp[e]
        st_v[pl.ds((E + e) * SC_NL, SC_NL)] = sl[e]
    for e in range(E):
        pltpu.sync_copy(wt_v.at[pl.ds(e * TPW, TPW)],
                        wt_hbm.at[pl.ds(e * N + base, TPW)])
    pltpu.sync_copy(st_v, st_hbm.at[pl.ds(wid * 2 * E * SC_NL, 2 * E * SC_NL)])


def _stage_b_kernel(ef_ref, wtt_ref, st_ref, gexp_ref, wup_ref,
                    out_ref, aux_ref):
    wt = wtt_ref[...].T
    act = _b16(_gelu_exact(ef_ref[...])).astype(F32)
    u = act * _dotx(wt, gexp_ref[...])
    out_ref[...] = _dotb(_b16(u), wup_ref[...])
    st = st_ref[...]
    s2 = jnp.sum(st, axis=2)
    s1 = jnp.sum(s2, axis=0, keepdims=True)
    aux_ref[...] = (jnp.sum(s1[:, :E] * s1[:, E:]) * (E / (N * N))).reshape(1, 1)


def kernel(x, w_down, expert_pos_embed, global_w, global_b, in_proj_w,
           in_proj_b, out_proj_w, out_proj_b, norm_g, norm_b, q_w1, q_b1,
           q_ln_g, q_ln_b, q_w2, q_b2, k_w, k_b, w_up):
    xf = x.reshape(N, DM)
    wdt = w_down.T.astype(BF16)
    pos = expert_pos_embed.reshape(1, E * DL)
    ipw_t = in_proj_w.T.astype(BF16)
    opw_t = out_proj_w.T.astype(BF16)
    kwt = k_w.T.astype(BF16)
    gwt = global_w.T.astype(BF16)
    qw1t = q_w1.T.astype(BF16)
    qw2t = q_w2.T.astype(BF16)
    wup = w_up.reshape(E * DL, DM).astype(BF16)

    def row(vec):
        return vec.reshape(1, -1)

    r576 = jnp.arange(S * DL)
    jj = r576 // DL
    d = r576 % DL
    a36 = jnp.arange(H * S)
    cols36 = (d // DH) * S + jj
    G2 = (cols36[:, None] == a36[None, :]).astype(BF16)
    Gss = ((a36[:, None] // S) == (a36[None, :] // S)).astype(F32)
    P36 = (((a36 % S)[:, None] == jj[None, :])
           & ((a36 // S)[:, None] == (d[None, :] // DH))).astype(BF16)
    Gexp = (jnp.arange(E)[:, None] == (jnp.arange(E * DL)[None, :] // DL)).astype(F32)

    gc, rq = pl.pallas_call(
        _ctx_kernel,
        grid=(B,),
        in_specs=[
            pl.BlockSpec((1, T, DM), lambda i: (i, 0, 0)),
            pl.BlockSpec((DM, DL), lambda i: (0, 0)),
            pl.BlockSpec((1, DL), lambda i: (0, 0)),
            pl.BlockSpec((DL, DL), lambda i: (0, 0)),
            pl.BlockSpec((1, DL), lambda i: (0, 0)),
            pl.BlockSpec((1, DL), lambda i: (0, 0)),
            pl.BlockSpec((1, DL), lambda i: (0, 0)),
            pl.BlockSpec((DL, DL), lambda i: (0, 0)),
            pl.BlockSpec((1, DL), lambda i: (0, 0)),
        ],
        out_specs=[
            pl.BlockSpec((1, 1, DL), lambda i: (i, 0, 0)),
            pl.BlockSpec((1, 1, DL), lambda i: (i, 0, 0)),
        ],
        out_shape=[
            jax.ShapeDtypeStruct((B, 1, DL), F32),
            jax.ShapeDtypeStruct((B, 1, DL), F32),
        ],
        compiler_params=pltpu.CompilerParams(
            dimension_semantics=("arbitrary",)),
    )(x, gwt, row(global_b), qw1t, row(q_b1), row(q_ln_g), row(q_ln_b),
      qw2t, row(q_b2))

    ef, lgt = pl.pallas_call(
        _main_kernel,
        grid=(NBLK,),
        in_specs=[
            pl.BlockSpec((BLK, DM), lambda i: (i, 0)),
            pl.BlockSpec((DM, E * DL), lambda i: (0, 0)),
            pl.BlockSpec((1, E * DL), lambda i: (0, 0)),
            pl.BlockSpec((1, 1, DL), lambda i: (i // TPB, 0, 0)),
            pl.BlockSpec((1, 1, DL), lambda i: (i // TPB, 0, 0)),
            pl.BlockSpec((DL, 3 * DL), lambda i: (0, 0)),
            pl.BlockSpec((1, 3 * DL), lambda i: (0, 0)),
            pl.BlockSpec((DL, DL), lambda i: (0, 0)),
            pl.BlockSpec((1, DL), lambda i: (0, 0)),
            pl.BlockSpec((1, DL), lambda i: (0, 0)),
            pl.BlockSpec((1, DL), lambda i: (0, 0)),
            pl.BlockSpec((DL, DL), lambda i: (0, 0)),
            pl.BlockSpec((1, DL), lambda i: (0, 0)),
            pl.BlockSpec((S * DL, H * S), lambda i: (0, 0)),
            pl.BlockSpec((H * S, H * S), lambda i: (0, 0)),
            pl.BlockSpec((H * S, S * DL), lambda i: (0, 0)),
        ],
        out_specs=[
            pl.BlockSpec((BLK, E * DL), lambda i: (i, 0)),
            pl.BlockSpec((E, BLK), lambda i: (0, i)),
        ],
        out_shape=[
            jax.ShapeDtypeStruct((N, E * DL), F32),
            jax.ShapeDtypeStruct((E, N), F32),
        ],
        compiler_params=pltpu.CompilerParams(
            dimension_semantics=("arbitrary",)),
    )(xf, wdt, pos, gc, rq, ipw_t, row(in_proj_b), opw_t, row(out_proj_b),
      row(norm_g), row(norm_b), kwt, row(k_b), G2, Gss, P36)

    sc_route = pl.kernel(
        _sc_route,
        mesh=plsc.VectorSubcoreMesh(core_axis_name="c", subcore_axis_name="s"),
        out_type=[
            jax.ShapeDtypeStruct((E * N,), F32),
            jax.ShapeDtypeStruct((NW * 2 * E * SC_NL,), F32),
        ],
        scratch_types=[
            pltpu.VMEM((E * TPW,), F32),
            pltpu.VMEM((E * TPW,), F32),
            pltpu.VMEM((2 * E * SC_NL,), F32),
        ],
    )
    wtt_flat, st_flat = sc_route(lgt.reshape(E * N))
    wtt = wtt_flat.reshape(E, N)
    st = st_flat.reshape(NW, 2 * E, SC_NL)

    out, aux = pl.pallas_call(
        _stage_b_kernel,
        grid=(NBLK,),
        in_specs=[
            pl.BlockSpec((BLK, E * DL), lambda i: (i, 0)),
            pl.BlockSpec((E, BLK), lambda i: (0, i)),
            pl.BlockSpec((NW, 2 * E, SC_NL), lambda i: (0, 0, 0)),
            pl.BlockSpec((E, E * DL), lambda i: (0, 0)),
            pl.BlockSpec((E * DL, DM), lambda i: (0, 0)),
        ],
        out_specs=[
            pl.BlockSpec((BLK, DM), lambda i: (i, 0)),
            pl.BlockSpec((1, 1), lambda i: (0, 0)),
        ],
        out_shape=[
            jax.ShapeDtypeStruct((N, DM), F32),
            jax.ShapeDtypeStruct((1, 1), F32),
        ],
        compiler_params=pltpu.CompilerParams(
            dimension_semantics=("arbitrary",)),
    )(ef, wtt, st, Gexp, wup)

    return out.reshape(B, T, DM), aux[0, 0]
